# baseline (speedup 1.0000x reference)
import jax
import jax.numpy as jnp
from jax.experimental import pallas as pl
from jax.experimental.pallas import tpu as pltpu

_M1, _M2 = 4, 5
_D = 128
_E = 128
_B = 16384
_NC = 1000
_BT = 1024


def _fused_body(x1_ref, x2_ref, w1_ref, b1_ref, m1_ref, w2_ref, b2_ref,
                m2_ref, wll_ref, bll_ref, out_ref, out1_ref):
    acc1 = jnp.zeros((x1_ref.shape[1], _E), dtype=jnp.float32)
    for m in range(_M1):
        wm = w1_ref[m] * m1_ref[m][None, :]
        acc1 = acc1 + jnp.dot(x1_ref[m], wm, preferred_element_type=jnp.float32)
    bsel1 = jnp.sum(b1_ref[...] * m1_ref[...], axis=0, keepdims=True)
    h1 = jnp.maximum(acc1 + bsel1, 0.0)
    out1_ref[...] = h1

    acc2 = jnp.dot(h1, w2_ref[_M2 - 1] * m2_ref[_M2 - 1][None, :],
                   preferred_element_type=jnp.float32)
    for m in range(_M1):
        wm = w2_ref[m] * m2_ref[m][None, :]
        acc2 = acc2 + jnp.dot(x2_ref[m], wm, preferred_element_type=jnp.float32)
    bsel2 = jnp.sum(b2_ref[...] * m2_ref[...], axis=0, keepdims=True)
    h2 = jnp.maximum(acc2 + bsel2, 0.0)

    out_ref[...] = jnp.dot(h2, wll_ref[...],
                           preferred_element_type=jnp.float32) + bll_ref[...]


def kernel(outputs1, outputs2, available, W1, b1, W2, b2, W_ll2, b_ll2):
    del available
    p1 = jnp.ones((1, _M1), dtype=jnp.float32) / _M1
    p2 = jnp.ones((1, _M2), dtype=jnp.float32) / _M2
    idx1 = jax.random.categorical(jax.random.key(42), jnp.log(p1)[0],
                                  shape=(1, _E))
    idx2 = jax.random.categorical(jax.random.key(43), jnp.log(p2)[0],
                                  shape=(1, _E))
    mask1 = jnp.transpose(jax.nn.one_hot(idx1, _M1, dtype=jnp.float32),
                          (0, 2, 1))[0]
    mask2 = jnp.transpose(jax.nn.one_hot(idx2, _M2, dtype=jnp.float32),
                          (0, 2, 1))[0]

    grid = (_B // _BT,)
    full = lambda *shape: pl.BlockSpec(shape, lambda i: (0,) * len(shape))
    out, out1 = pl.pallas_call(
        _fused_body,
        grid=grid,
        in_specs=[
            pl.BlockSpec((_M1, _BT, _D), lambda i: (0, i, 0)),
            pl.BlockSpec((_M1, _BT, _D), lambda i: (0, i, 0)),
            full(_M1, _D, _E),
            full(_M1, _E),
            full(_M1, _E),
            full(_M2, _D, _E),
            full(_M2, _E),
            full(_M2, _E),
            full(_E, _NC),
            full(1, _NC),
        ],
        out_specs=[
            pl.BlockSpec((_BT, _NC), lambda i: (i, 0)),
            pl.BlockSpec((_BT, _E), lambda i: (i, 0)),
        ],
        out_shape=[
            jax.ShapeDtypeStruct((_B, _NC), jnp.float32),
            jax.ShapeDtypeStruct((_B, _E), jnp.float32),
        ],
        compiler_params=pltpu.CompilerParams(
            dimension_semantics=("parallel",)),
    )(outputs1, outputs2, W1, b1, mask1, W2, b2, mask2, W_ll2,
      b_ll2.reshape(1, _NC))
    return (out, out1)


# speedup vs baseline: 1.0110x; 1.0110x over previous
import jax
import jax.numpy as jnp
import numpy as np
from jax.experimental import pallas as pl
from jax.experimental.pallas import tpu as pltpu

_M1, _M2 = 4, 5
_D = 128
_E = 128
_B = 16384
_NC = 1000
_BT = 1024


def _selection_masks():
    p1 = jnp.ones((1, _M1), dtype=jnp.float32) / _M1
    p2 = jnp.ones((1, _M2), dtype=jnp.float32) / _M2
    idx1 = jax.random.categorical(jax.random.key(42), jnp.log(p1)[0],
                                  shape=(1, _E))
    idx2 = jax.random.categorical(jax.random.key(43), jnp.log(p2)[0],
                                  shape=(1, _E))
    m1 = jnp.transpose(jax.nn.one_hot(idx1, _M1, dtype=jnp.float32),
                       (0, 2, 1))[0]
    m2 = jnp.transpose(jax.nn.one_hot(idx2, _M2, dtype=jnp.float32),
                       (0, 2, 1))[0]
    return np.asarray(m1), np.asarray(m2)


_MASK1, _MASK2 = _selection_masks()


def _fused_body(x1_ref, x2_ref, w1_ref, b1_ref, m1_ref, w2_ref, b2_ref,
                m2_ref, wll_ref, bll_ref, out_ref, out1_ref):
    acc1 = jnp.zeros((x1_ref.shape[1], _E), dtype=jnp.float32)
    for m in range(_M1):
        wm = w1_ref[m] * m1_ref[m][None, :]
        acc1 = acc1 + jnp.dot(x1_ref[m], wm, preferred_element_type=jnp.float32)
    bsel1 = jnp.sum(b1_ref[...] * m1_ref[...], axis=0, keepdims=True)
    h1 = jnp.maximum(acc1 + bsel1, 0.0)
    out1_ref[...] = h1

    acc2 = jnp.dot(h1, w2_ref[_M2 - 1] * m2_ref[_M2 - 1][None, :],
                   preferred_element_type=jnp.float32)
    for m in range(_M1):
        wm = w2_ref[m] * m2_ref[m][None, :]
        acc2 = acc2 + jnp.dot(x2_ref[m], wm, preferred_element_type=jnp.float32)
    bsel2 = jnp.sum(b2_ref[...] * m2_ref[...], axis=0, keepdims=True)
    h2 = jnp.maximum(acc2 + bsel2, 0.0)

    out_ref[...] = jnp.dot(h2, wll_ref[...],
                           preferred_element_type=jnp.float32) + bll_ref[...]


def kernel(outputs1, outputs2, available, W1, b1, W2, b2, W_ll2, b_ll2):
    del available
    mask1 = jnp.asarray(_MASK1)
    mask2 = jnp.asarray(_MASK2)

    grid = (_B // _BT,)
    full = lambda *shape: pl.BlockSpec(shape, lambda i: (0,) * len(shape))
    out, out1 = pl.pallas_call(
        _fused_body,
        grid=grid,
        in_specs=[
            pl.BlockSpec((_M1, _BT, _D), lambda i: (0, i, 0)),
            pl.BlockSpec((_M1, _BT, _D), lambda i: (0, i, 0)),
            full(_M1, _D, _E),
            full(_M1, _E),
            full(_M1, _E),
            full(_M2, _D, _E),
            full(_M2, _E),
            full(_M2, _E),
            full(_E, _NC),
            full(1, _NC),
        ],
        out_specs=[
            pl.BlockSpec((_BT, _NC), lambda i: (i, 0)),
            pl.BlockSpec((_BT, _E), lambda i: (i, 0)),
        ],
        out_shape=[
            jax.ShapeDtypeStruct((_B, _NC), jnp.float32),
            jax.ShapeDtypeStruct((_B, _E), jnp.float32),
        ],
        compiler_params=pltpu.CompilerParams(
            dimension_semantics=("parallel",)),
    )(outputs1, outputs2, W1, b1, mask1, W2, b2, mask2, W_ll2,
      b_ll2.reshape(1, _NC))
    return (out, out1)


# speedup vs baseline: 2.1322x; 2.1090x over previous
import jax
import jax.numpy as jnp
import numpy as np
from jax.experimental import pallas as pl
from jax.experimental.pallas import tpu as pltpu

_M1, _M2 = 4, 5
_D = 128
_E = 128
_B = 16384
_NC = 1000
_BT = 1024


_IDX1 = np.array([
    1, 3, 3, 0, 3, 3, 1, 1, 1, 0, 1, 3, 0, 1, 3, 2, 2, 0, 3, 3, 1, 3, 0, 3,
    2, 1, 1, 3, 1, 2, 3, 0, 3, 2, 3, 1, 3, 0, 3, 2, 3, 1, 2, 1, 0, 1, 3, 3,
    2, 2, 3, 2, 3, 1, 3, 2, 2, 2, 1, 2, 3, 2, 1, 1, 2, 1, 3, 3, 2, 2, 1, 3,
    2, 0, 0, 3, 2, 1, 3, 1, 0, 3, 0, 2, 1, 1, 1, 2, 1, 0, 3, 3, 0, 1, 1, 2,
    3, 0, 0, 1, 0, 1, 3, 1, 2, 2, 1, 3, 3, 0, 1, 0, 0, 2, 2, 3, 3, 2, 1, 1,
    2, 2, 3, 0, 1, 0, 2, 2], dtype=np.int32)
_IDX2 = np.array([
    2, 1, 4, 4, 2, 2, 4, 2, 0, 4, 3, 0, 1, 0, 1, 1, 4, 3, 4, 2, 4, 1, 2, 4,
    4, 1, 3, 1, 0, 3, 0, 3, 0, 1, 4, 0, 2, 1, 0, 2, 1, 0, 0, 0, 1, 1, 2, 0,
    3, 4, 1, 3, 4, 3, 3, 3, 0, 3, 3, 2, 4, 1, 0, 1, 4, 3, 2, 1, 2, 4, 0, 3,
    0, 0, 3, 2, 2, 0, 3, 4, 1, 2, 1, 3, 4, 4, 3, 0, 4, 1, 4, 2, 3, 1, 2, 4,
    1, 3, 2, 2, 1, 0, 1, 4, 4, 2, 4, 2, 3, 2, 2, 1, 2, 0, 3, 0, 4, 2, 0, 2,
    0, 3, 1, 2, 0, 2, 2, 0], dtype=np.int32)


def _selection_masks():
    m1 = (_IDX1[None, :] == np.arange(_M1)[:, None]).astype(np.float32)
    m2 = (_IDX2[None, :] == np.arange(_M2)[:, None]).astype(np.float32)
    return m1, m2


def _fused_body(x1_ref, x2_ref, w1_ref, b1_ref, m1_ref, w2_ref, b2_ref,
                m2_ref, wllt_ref, bllt_ref, out_ref, out1_ref):
    acc1 = jnp.zeros((x1_ref.shape[1], _E), dtype=jnp.float32)
    for m in range(_M1):
        wm = w1_ref[m] * m1_ref[m][None, :]
        acc1 = acc1 + jnp.dot(x1_ref[m], wm, preferred_element_type=jnp.float32)
    bsel1 = jnp.sum(b1_ref[...] * m1_ref[...], axis=0, keepdims=True)
    h1 = jnp.maximum(acc1 + bsel1, 0.0)
    out1_ref[...] = h1

    acc2 = jnp.dot(h1, w2_ref[_M2 - 1] * m2_ref[_M2 - 1][None, :],
                   preferred_element_type=jnp.float32)
    for m in range(_M1):
        wm = w2_ref[m] * m2_ref[m][None, :]
        acc2 = acc2 + jnp.dot(x2_ref[m], wm, preferred_element_type=jnp.float32)
    bsel2 = jnp.sum(b2_ref[...] * m2_ref[...], axis=0, keepdims=True)
    h2 = jnp.maximum(acc2 + bsel2, 0.0)

    out_ref[...] = jax.lax.dot_general(
        wllt_ref[...], h2, (((1,), (1,)), ((), ())),
        preferred_element_type=jnp.float32) + bllt_ref[...]


def kernel(outputs1, outputs2, available, W1, b1, W2, b2, W_ll2, b_ll2):
    del available
    m1np, m2np = _selection_masks()
    mask1 = jnp.asarray(m1np)
    mask2 = jnp.asarray(m2np)

    grid = (_B // _BT,)
    full = lambda *shape: pl.BlockSpec(shape, lambda i: (0,) * len(shape))
    out, out1 = pl.pallas_call(
        _fused_body,
        grid=grid,
        in_specs=[
            pl.BlockSpec((_M1, _BT, _D), lambda i: (0, i, 0)),
            pl.BlockSpec((_M1, _BT, _D), lambda i: (0, i, 0)),
            full(_M1, _D, _E),
            full(_M1, _E),
            full(_M1, _E),
            full(_M2, _D, _E),
            full(_M2, _E),
            full(_M2, _E),
            full(_NC, _E),
            full(_NC, 1),
        ],
        out_specs=[
            pl.BlockSpec((_NC, _BT), lambda i: (0, i)),
            pl.BlockSpec((_BT, _E), lambda i: (i, 0)),
        ],
        out_shape=[
            jax.ShapeDtypeStruct((_NC, _B), jnp.float32),
            jax.ShapeDtypeStruct((_B, _E), jnp.float32),
        ],
        compiler_params=pltpu.CompilerParams(
            dimension_semantics=("parallel",)),
    )(outputs1, outputs2, W1, b1, mask1, W2, b2, mask2, W_ll2.T,
      b_ll2.reshape(_NC, 1))
    return (jnp.transpose(out), out1)


# speedup vs baseline: 2.2329x; 1.0472x over previous
import jax
import jax.numpy as jnp
import numpy as np
from jax.experimental import pallas as pl
from jax.experimental.pallas import tpu as pltpu

_M1, _M2 = 4, 5
_D = 128
_E = 128
_B = 16384
_NC = 1000
_BT = 2048


_IDX1 = np.array([
    1, 3, 3, 0, 3, 3, 1, 1, 1, 0, 1, 3, 0, 1, 3, 2, 2, 0, 3, 3, 1, 3, 0, 3,
    2, 1, 1, 3, 1, 2, 3, 0, 3, 2, 3, 1, 3, 0, 3, 2, 3, 1, 2, 1, 0, 1, 3, 3,
    2, 2, 3, 2, 3, 1, 3, 2, 2, 2, 1, 2, 3, 2, 1, 1, 2, 1, 3, 3, 2, 2, 1, 3,
    2, 0, 0, 3, 2, 1, 3, 1, 0, 3, 0, 2, 1, 1, 1, 2, 1, 0, 3, 3, 0, 1, 1, 2,
    3, 0, 0, 1, 0, 1, 3, 1, 2, 2, 1, 3, 3, 0, 1, 0, 0, 2, 2, 3, 3, 2, 1, 1,
    2, 2, 3, 0, 1, 0, 2, 2], dtype=np.int32)
_IDX2 = np.array([
    2, 1, 4, 4, 2, 2, 4, 2, 0, 4, 3, 0, 1, 0, 1, 1, 4, 3, 4, 2, 4, 1, 2, 4,
    4, 1, 3, 1, 0, 3, 0, 3, 0, 1, 4, 0, 2, 1, 0, 2, 1, 0, 0, 0, 1, 1, 2, 0,
    3, 4, 1, 3, 4, 3, 3, 3, 0, 3, 3, 2, 4, 1, 0, 1, 4, 3, 2, 1, 2, 4, 0, 3,
    0, 0, 3, 2, 2, 0, 3, 4, 1, 2, 1, 3, 4, 4, 3, 0, 4, 1, 4, 2, 3, 1, 2, 4,
    1, 3, 2, 2, 1, 0, 1, 4, 4, 2, 4, 2, 3, 2, 2, 1, 2, 0, 3, 0, 4, 2, 0, 2,
    0, 3, 1, 2, 0, 2, 2, 0], dtype=np.int32)


def _selection_masks():
    m1 = (_IDX1[None, :] == np.arange(_M1)[:, None]).astype(np.float32)
    m2 = (_IDX2[None, :] == np.arange(_M2)[:, None]).astype(np.float32)
    return m1, m2


def _fused_body(x1_ref, x2_ref, w1_ref, b1_ref, m1_ref, w2_ref, b2_ref,
                m2_ref, wllt_ref, bllt_ref, out_ref, out1_ref):
    acc1 = jnp.zeros((x1_ref.shape[1], _E), dtype=jnp.float32)
    for m in range(_M1):
        wm = w1_ref[m] * m1_ref[m][None, :]
        acc1 = acc1 + jnp.dot(x1_ref[m], wm, preferred_element_type=jnp.float32)
    bsel1 = jnp.sum(b1_ref[...] * m1_ref[...], axis=0, keepdims=True)
    h1 = jnp.maximum(acc1 + bsel1, 0.0)
    out1_ref[...] = h1

    acc2 = jnp.dot(h1, w2_ref[_M2 - 1] * m2_ref[_M2 - 1][None, :],
                   preferred_element_type=jnp.float32)
    for m in range(_M1):
        wm = w2_ref[m] * m2_ref[m][None, :]
        acc2 = acc2 + jnp.dot(x2_ref[m], wm, preferred_element_type=jnp.float32)
    bsel2 = jnp.sum(b2_ref[...] * m2_ref[...], axis=0, keepdims=True)
    h2 = jnp.maximum(acc2 + bsel2, 0.0)

    out_ref[...] = jax.lax.dot_general(
        wllt_ref[...], h2, (((1,), (1,)), ((), ())),
        preferred_element_type=jnp.float32) + bllt_ref[...]


def kernel(outputs1, outputs2, available, W1, b1, W2, b2, W_ll2, b_ll2):
    del available
    m1np, m2np = _selection_masks()
    mask1 = jnp.asarray(m1np)
    mask2 = jnp.asarray(m2np)

    grid = (_B // _BT,)
    full = lambda *shape: pl.BlockSpec(shape, lambda i: (0,) * len(shape))
    out, out1 = pl.pallas_call(
        _fused_body,
        grid=grid,
        in_specs=[
            pl.BlockSpec((_M1, _BT, _D), lambda i: (0, i, 0)),
            pl.BlockSpec((_M1, _BT, _D), lambda i: (0, i, 0)),
            full(_M1, _D, _E),
            full(_M1, _E),
            full(_M1, _E),
            full(_M2, _D, _E),
            full(_M2, _E),
            full(_M2, _E),
            full(_NC, _E),
            full(_NC, 1),
        ],
        out_specs=[
            pl.BlockSpec((_NC, _BT), lambda i: (0, i)),
            pl.BlockSpec((_BT, _E), lambda i: (i, 0)),
        ],
        out_shape=[
            jax.ShapeDtypeStruct((_NC, _B), jnp.float32),
            jax.ShapeDtypeStruct((_B, _E), jnp.float32),
        ],
        compiler_params=pltpu.CompilerParams(
            dimension_semantics=("parallel",)),
    )(outputs1, outputs2, W1, b1, mask1, W2, b2, mask2, W_ll2.T,
      b_ll2.reshape(_NC, 1))
    return (jnp.transpose(out), out1)


# speedup vs baseline: 2.3590x; 1.0565x over previous
import jax
import jax.numpy as jnp
import numpy as np
from jax.experimental import pallas as pl
from jax.experimental.pallas import tpu as pltpu

_M1, _M2 = 4, 5
_D = 128
_E = 128
_B = 16384
_NC = 1000
_BT = 2048


_IDX1 = np.array([
    1, 3, 3, 0, 3, 3, 1, 1, 1, 0, 1, 3, 0, 1, 3, 2, 2, 0, 3, 3, 1, 3, 0, 3,
    2, 1, 1, 3, 1, 2, 3, 0, 3, 2, 3, 1, 3, 0, 3, 2, 3, 1, 2, 1, 0, 1, 3, 3,
    2, 2, 3, 2, 3, 1, 3, 2, 2, 2, 1, 2, 3, 2, 1, 1, 2, 1, 3, 3, 2, 2, 1, 3,
    2, 0, 0, 3, 2, 1, 3, 1, 0, 3, 0, 2, 1, 1, 1, 2, 1, 0, 3, 3, 0, 1, 1, 2,
    3, 0, 0, 1, 0, 1, 3, 1, 2, 2, 1, 3, 3, 0, 1, 0, 0, 2, 2, 3, 3, 2, 1, 1,
    2, 2, 3, 0, 1, 0, 2, 2], dtype=np.int32)
_IDX2 = np.array([
    2, 1, 4, 4, 2, 2, 4, 2, 0, 4, 3, 0, 1, 0, 1, 1, 4, 3, 4, 2, 4, 1, 2, 4,
    4, 1, 3, 1, 0, 3, 0, 3, 0, 1, 4, 0, 2, 1, 0, 2, 1, 0, 0, 0, 1, 1, 2, 0,
    3, 4, 1, 3, 4, 3, 3, 3, 0, 3, 3, 2, 4, 1, 0, 1, 4, 3, 2, 1, 2, 4, 0, 3,
    0, 0, 3, 2, 2, 0, 3, 4, 1, 2, 1, 3, 4, 4, 3, 0, 4, 1, 4, 2, 3, 1, 2, 4,
    1, 3, 2, 2, 1, 0, 1, 4, 4, 2, 4, 2, 3, 2, 2, 1, 2, 0, 3, 0, 4, 2, 0, 2,
    0, 3, 1, 2, 0, 2, 2, 0], dtype=np.int32)


def _selection_masks():
    m1 = (_IDX1[None, :] == np.arange(_M1)[:, None]).astype(np.float32)
    m2 = (_IDX2[None, :] == np.arange(_M2)[:, None]).astype(np.float32)
    return m1, m2


def _fused_body(x1_ref, x2_ref, w1_ref, b1_ref, m1_ref, w2_ref, b2_ref,
                m2_ref, wllt_ref, bllt_ref, out_ref, out1_ref):
    acc1 = jnp.zeros((x1_ref.shape[1], _E), dtype=jnp.float32)
    for m in range(_M1):
        wm = w1_ref[m] * m1_ref[m][None, :]
        acc1 = acc1 + jnp.dot(x1_ref[m], wm, preferred_element_type=jnp.float32)
    bsel1 = jnp.sum(b1_ref[...] * m1_ref[...], axis=0, keepdims=True)
    h1 = jnp.maximum(acc1 + bsel1, 0.0)
    out1_ref[...] = h1

    acc2 = jnp.dot(h1, w2_ref[_M2 - 1] * m2_ref[_M2 - 1][None, :],
                   preferred_element_type=jnp.float32)
    for m in range(_M1):
        wm = w2_ref[m] * m2_ref[m][None, :]
        acc2 = acc2 + jnp.dot(x2_ref[m], wm, preferred_element_type=jnp.float32)
    bsel2 = jnp.sum(b2_ref[...] * m2_ref[...], axis=0, keepdims=True)
    h2 = jnp.maximum(acc2 + bsel2, 0.0)

    bll = bllt_ref[...].reshape(_NC, 1)
    out_ref[...] = jax.lax.dot_general(
        wllt_ref[...], h2, (((1,), (1,)), ((), ())),
        preferred_element_type=jnp.float32) + bll


def kernel(outputs1, outputs2, available, W1, b1, W2, b2, W_ll2, b_ll2):
    del available
    m1np, m2np = _selection_masks()
    mask1 = jnp.asarray(m1np)
    mask2 = jnp.asarray(m2np)

    grid = (_B // _BT,)
    full = lambda *shape: pl.BlockSpec(shape, lambda i: (0,) * len(shape))
    out, out1 = pl.pallas_call(
        _fused_body,
        grid=grid,
        in_specs=[
            pl.BlockSpec((_M1, _BT, _D), lambda i: (0, i, 0)),
            pl.BlockSpec((_M1, _BT, _D), lambda i: (0, i, 0)),
            full(_M1, _D, _E),
            full(_M1, _E),
            full(_M1, _E),
            full(_M2, _D, _E),
            full(_M2, _E),
            full(_M2, _E),
            full(_NC, _E),
            full(_NC),
        ],
        out_specs=[
            pl.BlockSpec((_NC, _BT), lambda i: (0, i)),
            pl.BlockSpec((_BT, _E), lambda i: (i, 0)),
        ],
        out_shape=[
            jax.ShapeDtypeStruct((_NC, _B), jnp.float32),
            jax.ShapeDtypeStruct((_B, _E), jnp.float32),
        ],
        compiler_params=pltpu.CompilerParams(
            dimension_semantics=("parallel",),
            vmem_limit_bytes=100 * 1024 * 1024),
    )(outputs1, outputs2, W1, b1, mask1, W2, b2, mask2, W_ll2.T, b_ll2)
    return (jnp.transpose(out), out1)
